# initial kernel scaffold (unmeasured)
import jax
import jax.numpy as jnp
from jax import lax
from jax.experimental import pallas as pl
from jax.experimental.pallas import tpu as pltpu

N_DEV = 4
F8 = jnp.float8_e4m3fn
XCH = 256
WCH = 256
HALF = 512
KC = 1024


def kernel(x, w_mat, scale_x, scale_w):
    m_sh, k = x.shape
    n = w_mat.shape[1]
    n_sh = n // N_DEV

    def body(x_hbm, w_hbm, sx_ref, sw_ref, out_hbm,
             xg, w8, xstage, wstage, acc,
             xsems, wsems, osems, send_sems, recv_sems):
        my = lax.axis_index("i")
        right = lax.rem(my + 1, N_DEV)
        left = lax.rem(my + N_DEV - 1, N_DEV)

        barrier = pltpu.get_barrier_semaphore()
        for nbr in (left, right):
            pl.semaphore_signal(barrier, inc=1, device_id=(nbr,),
                                device_id_type=pltpu.DeviceIdType.MESH)
        pl.semaphore_wait(barrier, 2)

        def x_copy(c):
            return pltpu.make_async_copy(
                x_hbm.at[pl.ds(c * XCH, XCH), :],
                xstage.at[c % 2], xsems.at[c % 2])

        def w_copy(t):
            return pltpu.make_async_copy(
                w_hbm.at[pl.ds(t * WCH, WCH), pl.ds(my * n_sh, n_sh)],
                wstage.at[t % 2], wsems.at[t % 2])

        n_xch = m_sh // XCH
        n_wch = k // WCH

        x_copy(0).start()
        x_copy(1).start()
        for c in range(n_xch):
            x_copy(c).wait()
            xg[0, pl.ds(c * XCH, XCH), :] = xstage[c % 2].astype(F8)
            if c + 2 < n_xch:
                x_copy(c + 2).start()

        def remote(src, dst, sem_idx, target):
            return pltpu.make_async_remote_copy(
                src_ref=src, dst_ref=dst,
                send_sem=send_sems.at[sem_idx],
                recv_sem=recv_sems.at[sem_idx],
                device_id=(target,),
                device_id_type=pltpu.DeviceIdType.MESH)

        r1 = remote(xg.at[0], xg.at[3], 0, right)
        r1.start()
        l1 = remote(xg.at[0], xg.at[1], 1, left)
        l1.start()

        w_copy(0).start()
        w_copy(1).start()
        for t in range(n_wch):
            w_copy(t).wait()
            w8[pl.ds(t * WCH, WCH), :] = wstage[t % 2].astype(F8)
            if t + 2 < n_wch:
                w_copy(t + 2).start()

        s = sx_ref[0] * sw_ref[0]
        pending_out = []

        def gemm(slot):
            if pending_out:
                pending_out.pop().wait()
            for kc in range(k // KC):
                a = xg[slot, :, kc * KC:(kc + 1) * KC]
                part = lax.dot_general(
                    a, w8[kc * KC:(kc + 1) * KC, :],
                    (((1,), (0,)), ((), ())),
                    preferred_element_type=jnp.float32)
                if kc == 0:
                    acc[...] = part
                else:
                    acc[...] += part
            acc[...] = jnp.maximum(acc[...] * s, 0.0)
            origin = lax.rem(my + slot, N_DEV)
            cp = pltpu.make_async_copy(
                acc, out_hbm.at[pl.ds(origin * m_sh, m_sh), :],
                osems.at[0])
            cp.start()
            pending_out.append(cp)

        gemm(0)

        remote(xg.at[3], xg.at[3], 0, right).wait_recv()
        fr = remote(xg.at[3, pl.ds(0, HALF)],
                    xg.at[2, pl.ds(0, HALF)], 2, right)
        fr.start()
        gemm(3)

        remote(xg.at[1], xg.at[1], 1, left).wait_recv()
        fl = remote(xg.at[1, pl.ds(HALF, HALF)],
                    xg.at[2, pl.ds(HALF, HALF)], 3, left)
        fl.start()
        gemm(1)

        remote(xg.at[2, pl.ds(0, HALF)],
               xg.at[2, pl.ds(0, HALF)], 2, right).wait_recv()
        remote(xg.at[2, pl.ds(HALF, HALF)],
               xg.at[2, pl.ds(HALF, HALF)], 3, left).wait_recv()
        gemm(2)

        while pending_out:
            pending_out.pop().wait()
        r1.wait_send()
        l1.wait_send()
        fr.wait_send()
        fl.wait_send()

    out_shape = jax.ShapeDtypeStruct((N_DEV * m_sh, n_sh), jnp.float32)
    return pl.pallas_call(
        body,
        out_shape=out_shape,
        in_specs=[
            pl.BlockSpec(memory_space=pltpu.ANY),
            pl.BlockSpec(memory_space=pltpu.ANY),
            pl.BlockSpec(memory_space=pltpu.SMEM),
            pl.BlockSpec(memory_space=pltpu.SMEM),
        ],
        out_specs=pl.BlockSpec(memory_space=pltpu.ANY),
        scratch_shapes=[
            pltpu.VMEM((N_DEV, m_sh, k), F8),
            pltpu.VMEM((k, n_sh), F8),
            pltpu.VMEM((2, XCH, k), jnp.float32),
            pltpu.VMEM((2, WCH, n_sh), jnp.float32),
            pltpu.VMEM((m_sh, n_sh), jnp.float32),
            pltpu.SemaphoreType.DMA((2,)),
            pltpu.SemaphoreType.DMA((2,)),
            pltpu.SemaphoreType.DMA((1,)),
            pltpu.SemaphoreType.DMA((4,)),
            pltpu.SemaphoreType.DMA((4,)),
        ],
        compiler_params=pltpu.CompilerParams(collective_id=0),
    )(x, w_mat, scale_x, scale_w)


# baseline (device time: 125420 ns/iter reference)
import jax
import jax.numpy as jnp
from jax import lax
from jax.experimental import pallas as pl
from jax.experimental.pallas import tpu as pltpu

N_DEV = 4
F8 = jnp.float8_e4m3fn
XCH = 256
WCH = 256
HALF = 512
KC = 1024


def kernel(x, w_mat, scale_x, scale_w):
    m_sh, k = x.shape
    n = w_mat.shape[1]
    n_sh = n // N_DEV

    def body(x_hbm, w_hbm, sx_ref, sw_ref, out_hbm,
             xg, w8, xstage, wstage, acc,
             xsems, wsems, osems, send_sems, recv_sems):
        my = lax.axis_index("i")
        right = lax.rem(my + 1, N_DEV)
        left = lax.rem(my + N_DEV - 1, N_DEV)

        barrier = pltpu.get_barrier_semaphore()
        for nbr in (left, right):
            pl.semaphore_signal(barrier, inc=1, device_id=(nbr,),
                                device_id_type=pltpu.DeviceIdType.MESH)
        pl.semaphore_wait(barrier, 2)

        def x_copy(c):
            return pltpu.make_async_copy(
                x_hbm.at[pl.ds(c * XCH, XCH), :],
                xstage.at[c % 2], xsems.at[c % 2])

        def w_copy(t):
            return pltpu.make_async_copy(
                w_hbm.at[pl.ds(t * WCH, WCH), pl.ds(my * n_sh, n_sh)],
                wstage.at[t % 2], wsems.at[t % 2])

        n_xch = m_sh // XCH
        n_wch = k // WCH

        x_copy(0).start()
        x_copy(1).start()
        for c in range(n_xch):
            x_copy(c).wait()
            xg[0, pl.ds(c * XCH, XCH), :] = xstage[c % 2].astype(F8)
            if c + 2 < n_xch:
                x_copy(c + 2).start()

        def remote(src, dst, sem_idx, target):
            return pltpu.make_async_remote_copy(
                src_ref=src, dst_ref=dst,
                send_sem=send_sems.at[sem_idx],
                recv_sem=recv_sems.at[sem_idx],
                device_id=(target,),
                device_id_type=pltpu.DeviceIdType.MESH)

        r1 = remote(xg.at[0], xg.at[3], 0, right)
        r1.start()
        l1 = remote(xg.at[0], xg.at[1], 1, left)
        l1.start()

        w_copy(0).start()
        w_copy(1).start()
        for t in range(n_wch):
            w_copy(t).wait()
            w8[pl.ds(t * WCH, WCH), :] = wstage[t % 2].astype(F8)
            if t + 2 < n_wch:
                w_copy(t + 2).start()

        s = sx_ref[0] * sw_ref[0]
        pending_out = []

        def gemm(slot):
            if pending_out:
                pending_out.pop().wait()
            for kc in range(k // KC):
                a = xg[slot, :, kc * KC:(kc + 1) * KC]
                part = lax.dot_general(
                    a, w8[kc * KC:(kc + 1) * KC, :],
                    (((1,), (0,)), ((), ())),
                    preferred_element_type=jnp.float32)
                if kc == 0:
                    acc[...] = part
                else:
                    acc[...] += part
            acc[...] = jnp.maximum(acc[...] * s, 0.0)
            origin = lax.rem(my + slot, N_DEV)
            cp = pltpu.make_async_copy(
                acc, out_hbm.at[pl.ds(origin * m_sh, m_sh), :],
                osems.at[0])
            cp.start()
            pending_out.append(cp)

        gemm(0)

        remote(xg.at[3], xg.at[3], 0, right).wait_recv()
        fr = remote(xg.at[3, pl.ds(0, HALF)],
                    xg.at[2, pl.ds(0, HALF)], 2, right)
        fr.start()
        gemm(3)

        remote(xg.at[1], xg.at[1], 1, left).wait_recv()
        fl = remote(xg.at[1, pl.ds(HALF, HALF)],
                    xg.at[2, pl.ds(HALF, HALF)], 3, left)
        fl.start()
        gemm(1)

        remote(xg.at[2, pl.ds(0, HALF)],
               xg.at[2, pl.ds(0, HALF)], 2, right).wait_recv()
        remote(xg.at[2, pl.ds(HALF, HALF)],
               xg.at[2, pl.ds(HALF, HALF)], 3, left).wait_recv()
        gemm(2)

        while pending_out:
            pending_out.pop().wait()
        r1.wait_send()
        l1.wait_send()
        fr.wait_send()
        fl.wait_send()

    out_shape = jax.ShapeDtypeStruct((N_DEV * m_sh, n_sh), jnp.float32)
    return pl.pallas_call(
        body,
        out_shape=out_shape,
        in_specs=[
            pl.BlockSpec(memory_space=pl.ANY),
            pl.BlockSpec(memory_space=pl.ANY),
            pl.BlockSpec(memory_space=pltpu.SMEM),
            pl.BlockSpec(memory_space=pltpu.SMEM),
        ],
        out_specs=pl.BlockSpec(memory_space=pl.ANY),
        scratch_shapes=[
            pltpu.VMEM((N_DEV, m_sh, k), F8),
            pltpu.VMEM((k, n_sh), F8),
            pltpu.VMEM((2, XCH, k), jnp.float32),
            pltpu.VMEM((2, WCH, n_sh), jnp.float32),
            pltpu.VMEM((m_sh, n_sh), jnp.float32),
            pltpu.SemaphoreType.DMA((2,)),
            pltpu.SemaphoreType.DMA((2,)),
            pltpu.SemaphoreType.DMA((1,)),
            pltpu.SemaphoreType.DMA((4,)),
            pltpu.SemaphoreType.DMA((4,)),
        ],
        compiler_params=pltpu.CompilerParams(
            collective_id=0, vmem_limit_bytes=64 * 1024 * 1024),
    )(x, w_mat, scale_x, scale_w)


# device time: 113798 ns/iter; 1.1021x vs baseline; 1.1021x over previous
import jax
import jax.numpy as jnp
from jax import lax
from jax.experimental import pallas as pl
from jax.experimental.pallas import tpu as pltpu

N_DEV = 4
F8 = jnp.float8_e4m3fn
XCH = 256
WCH = 256
HALF = 512
NC = 1024


def kernel(x, w_mat, scale_x, scale_w):
    m_sh, k = x.shape
    n = w_mat.shape[1]
    n_sh = n // N_DEV

    def body(x_hbm, w_hbm, sx_ref, sw_ref, out_hbm,
             xg, w8, xstage, wstage, acc,
             xsems, wsems, osems, send_sems, recv_sems):
        my = lax.axis_index("i")
        right = lax.rem(my + 1, N_DEV)
        left = lax.rem(my + N_DEV - 1, N_DEV)

        barrier = pltpu.get_barrier_semaphore()
        for nbr in (left, right):
            pl.semaphore_signal(barrier, inc=1, device_id=(nbr,),
                                device_id_type=pltpu.DeviceIdType.MESH)
        pl.semaphore_wait(barrier, 2)

        def x_copy(c):
            return pltpu.make_async_copy(
                x_hbm.at[pl.ds(c * XCH, XCH), :],
                xstage.at[c % 2], xsems.at[c % 2])

        def w_copy(t):
            return pltpu.make_async_copy(
                w_hbm.at[pl.ds(t * WCH, WCH), pl.ds(my * n_sh, n_sh)],
                wstage.at[t % 2], wsems.at[t % 2])

        n_xch = m_sh // XCH
        n_wch = k // WCH

        def remote(src, dst, sem_idx, target):
            return pltpu.make_async_remote_copy(
                src_ref=src, dst_ref=dst,
                send_sem=send_sems.at[sem_idx],
                recv_sem=recv_sems.at[sem_idx],
                device_id=(target,),
                device_id_type=pltpu.DeviceIdType.MESH)

        TOP = pl.ds(0, HALF)
        BOT = pl.ds(HALF, HALF)
        Q = HALF // 2

        sends = [
            remote(xg.at[0, TOP], xg.at[3, TOP], 0, right),
            remote(xg.at[0, BOT], xg.at[3, BOT], 1, right),
            remote(xg.at[0, BOT], xg.at[1, BOT], 2, left),
            remote(xg.at[0, TOP], xg.at[1, TOP], 3, left),
        ]
        x_copy(0).start()
        x_copy(1).start()
        for c in range(n_xch):
            x_copy(c).wait()
            xg[0, pl.ds(c * XCH, XCH), :] = xstage[c % 2].astype(F8)
            if c + 2 < n_xch:
                x_copy(c + 2).start()
            if (c + 1) * XCH == HALF:
                sends[0].start()
        for rd in sends[1:]:
            rd.start()

        w_copy(0).start()
        w_copy(1).start()
        for t in range(n_wch):
            w_copy(t).wait()
            w8[pl.ds(t * WCH, WCH), :] = wstage[t % 2].astype(F8)
            if t + 2 < n_wch:
                w_copy(t + 2).start()

        s = sx_ref[0] * sw_ref[0]
        pending_out = []

        def gemm(slot):
            a = xg[slot]
            for nc in range(n_sh // NC):
                r = lax.dot_general(
                    a, w8[:, nc * NC:(nc + 1) * NC],
                    (((1,), (0,)), ((), ())),
                    preferred_element_type=jnp.float32)
                y = jnp.maximum(r * s, 0.0)
                if nc == 0 and pending_out:
                    pending_out.pop().wait()
                acc[:, pl.ds(nc * NC, NC)] = y
            origin = lax.rem(my + slot, N_DEV)
            cp = pltpu.make_async_copy(
                acc, out_hbm.at[pl.ds(origin * m_sh, m_sh), :],
                osems.at[0])
            cp.start()
            pending_out.append(cp)

        gemm(0)

        def quarter(i):
            return pl.ds(i * Q, Q)

        remote(xg.at[3, TOP], xg.at[3, TOP], 0, right).wait_recv()
        fwds = [remote(xg.at[3, quarter(q)], xg.at[2, quarter(q)],
                       4 + q, right) for q in range(2)]
        for rd in fwds[:2]:
            rd.start()
        remote(xg.at[3, BOT], xg.at[3, BOT], 1, right).wait_recv()
        gemm(3)

        remote(xg.at[1, BOT], xg.at[1, BOT], 2, left).wait_recv()
        fwds += [remote(xg.at[1, quarter(q)], xg.at[2, quarter(q)],
                        4 + q, left) for q in range(2, 4)]
        for rd in fwds[2:]:
            rd.start()
        remote(xg.at[1, TOP], xg.at[1, TOP], 3, left).wait_recv()
        gemm(1)

        diag_origin = lax.rem(my + 2, N_DEV)
        for qi, (row, sem) in enumerate([(0, 4), (2, 6), (1, 5), (3, 7)]):
            remote(xg.at[2, quarter(row)], xg.at[2, quarter(row)],
                   sem, right).wait_recv()
            a = xg[2, quarter(row)]
            for nc in range(n_sh // NC):
                r = lax.dot_general(
                    a, w8[:, nc * NC:(nc + 1) * NC],
                    (((1,), (0,)), ((), ())),
                    preferred_element_type=jnp.float32)
                y = jnp.maximum(r * s, 0.0)
                if qi == 0 and nc == 0 and pending_out:
                    pending_out.pop().wait()
                acc[quarter(row), pl.ds(nc * NC, NC)] = y
            cp = pltpu.make_async_copy(
                acc.at[quarter(row)],
                out_hbm.at[pl.ds(diag_origin * m_sh + row * Q, Q), :],
                osems.at[1])
            if qi > 0:
                pending_out.pop().wait()
            cp.start()
            pending_out.append(cp)

        while pending_out:
            pending_out.pop().wait()
        for rd in sends + fwds:
            rd.wait_send()

    out_shape = jax.ShapeDtypeStruct((N_DEV * m_sh, n_sh), jnp.float32)
    return pl.pallas_call(
        body,
        out_shape=out_shape,
        in_specs=[
            pl.BlockSpec(memory_space=pl.ANY),
            pl.BlockSpec(memory_space=pl.ANY),
            pl.BlockSpec(memory_space=pltpu.SMEM),
            pl.BlockSpec(memory_space=pltpu.SMEM),
        ],
        out_specs=pl.BlockSpec(memory_space=pl.ANY),
        scratch_shapes=[
            pltpu.VMEM((N_DEV, m_sh, k), F8),
            pltpu.VMEM((k, n_sh), F8),
            pltpu.VMEM((2, XCH, k), jnp.float32),
            pltpu.VMEM((2, WCH, n_sh), jnp.float32),
            pltpu.VMEM((m_sh, n_sh), jnp.float32),
            pltpu.SemaphoreType.DMA((2,)),
            pltpu.SemaphoreType.DMA((2,)),
            pltpu.SemaphoreType.DMA((2,)),
            pltpu.SemaphoreType.DMA((8,)),
            pltpu.SemaphoreType.DMA((8,)),
        ],
        compiler_params=pltpu.CompilerParams(
            collective_id=0, vmem_limit_bytes=64 * 1024 * 1024),
    )(x, w_mat, scale_x, scale_w)
